# Initial kernel scaffold; baseline (speedup 1.0000x reference)
import jax
import jax.numpy as jnp
from jax.experimental import pallas as pl


def kernel(flows, boxes):
    raise NotImplementedError("write your pallas kernel here")


# speedup vs baseline: 40.0182x; 40.0182x over previous
import jax
import jax.numpy as jnp
from jax.experimental import pallas as pl
from jax.experimental.pallas import tpu as pltpu

_OUT = 224
_WIN_Y = 240
_WIN_X = 384
_NBINS = 8


def _ve_kernel(boxes_ref, flows_ref, out_ref):
    i = pl.program_id(0)
    H = flows_ref.shape[2]
    W = flows_ref.shape[3]

    b = boxes_ref[i, 0].astype(jnp.int32)
    x1 = boxes_ref[i, 1]
    y1 = boxes_ref[i, 2]
    x2 = boxes_ref[i, 3]
    y2 = boxes_ref[i, 4]
    roi_w = jnp.maximum(x2 - x1, 1.0)
    roi_h = jnp.maximum(y2 - y1, 1.0)
    bin_h = roi_h / _OUT
    bin_w = roi_w / _OUT

    ys0 = jnp.clip(y1 + 0.5 * bin_h, 0.0, H - 1.0)
    xs0 = jnp.clip(x1 + 0.5 * bin_w, 0.0, W - 1.0)
    sy = jnp.minimum(
        (jnp.floor(ys0).astype(jnp.int32) // 8) * 8, H - _WIN_Y
    )
    sx = jnp.minimum(
        (jnp.floor(xs0).astype(jnp.int32) // 128) * 128, W - _WIN_X
    )

    jf = jax.lax.broadcasted_iota(jnp.int32, (_OUT, 1), 0).astype(jnp.float32)
    ys = jnp.clip(y1 + (jf + 0.5) * bin_h, 0.0, H - 1.0)
    xs = jnp.clip(x1 + (jf + 0.5) * bin_w, 0.0, W - 1.0)
    y0f = jnp.floor(ys)
    x0f = jnp.floor(xs)
    wy = ys - y0f
    wx = xs - x0f
    y0 = y0f.astype(jnp.int32)
    x0 = x0f.astype(jnp.int32)
    y1i = jnp.minimum(y0 + 1, H - 1)
    x1i = jnp.minimum(x0 + 1, W - 1)

    y0r = y0 - sy
    y1r = y1i - sy
    x0r = x0 - sx
    x1r = x1i - sx

    kky = jax.lax.broadcasted_iota(jnp.int32, (_OUT, _WIN_Y), 1)
    kkx = jax.lax.broadcasted_iota(jnp.int32, (_OUT, _WIN_X), 1)
    w_y = (kky == y0r).astype(jnp.float32) * (1.0 - wy) + (
        kky == y1r
    ).astype(jnp.float32) * wy
    w_x = (kkx == x0r).astype(jnp.float32) * (1.0 - wx) + (
        kkx == x1r
    ).astype(jnp.float32) * wx

    win0 = flows_ref[b, 0, pl.ds(sy, _WIN_Y), pl.ds(sx, _WIN_X)]
    win1 = flows_ref[b, 1, pl.ds(sy, _WIN_Y), pl.ds(sx, _WIN_X)]

    dn_rows = (((1,), (0,)), ((), ()))
    dn_cols = (((1,), (1,)), ((), ()))
    hp = jax.lax.Precision.HIGHEST
    ty0 = jax.lax.dot_general(
        w_y, win0, dn_rows, precision=hp, preferred_element_type=jnp.float32
    )
    ty1 = jax.lax.dot_general(
        w_y, win1, dn_rows, precision=hp, preferred_element_type=jnp.float32
    )
    u = jax.lax.dot_general(
        ty0, w_x, dn_cols, precision=hp, preferred_element_type=jnp.float32
    )
    v = jax.lax.dot_general(
        ty1, w_x, dn_cols, precision=hp, preferred_element_type=jnp.float32
    )

    mag = jnp.sqrt(u * u + v * v)

    su = u >= 0.0
    sv = v >= 0.0
    sm = jnp.abs(u) >= jnp.abs(v)
    nsu = jnp.logical_not(su)
    nsv = jnp.logical_not(sv)
    nsm = jnp.logical_not(sm)
    masks = [
        nsu & nsv & nsm,
        nsu & nsv & sm,
        nsu & sv & sm,
        nsu & sv & nsm,
        su & sv & nsm,
        su & sv & sm,
        su & nsv & sm,
        su & nsv & nsm,
    ]
    zeros = jnp.zeros_like(mag)
    vals = []
    for m in masks:
        s = jnp.sum(jnp.where(m, mag, zeros))
        c = jnp.sum(m.astype(jnp.float32))
        vals.append(jnp.where(c > 0.0, s / jnp.maximum(c, 1.0), 0.0))
    out_ref[i, :] = jnp.stack(vals)


def kernel(flows, boxes):
    K = boxes.shape[0]
    return pl.pallas_call(
        _ve_kernel,
        grid=(K,),
        in_specs=[
            pl.BlockSpec(memory_space=pltpu.SMEM),
            pl.BlockSpec(
                flows.shape, lambda i: (0, 0, 0, 0), memory_space=pltpu.VMEM
            ),
        ],
        out_specs=pl.BlockSpec((K, _NBINS), lambda i: (0, 0)),
        out_shape=jax.ShapeDtypeStruct((K, _NBINS), jnp.float32),
        compiler_params=pltpu.CompilerParams(
            dimension_semantics=("arbitrary",),
        ),
    )(boxes, flows)


# speedup vs baseline: 68.9080x; 1.7219x over previous
import jax
import jax.numpy as jnp
from jax.experimental import pallas as pl
from jax.experimental.pallas import tpu as pltpu

_OUT = 224
_WIN_Y = 240
_WIN_X = 384
_NBINS = 8


def _ve_kernel(boxes_ref, flows_ref, out_ref):
    i = pl.program_id(0)
    H = flows_ref.shape[2]
    W = flows_ref.shape[3]

    b = boxes_ref[i, 0].astype(jnp.int32)
    x1 = boxes_ref[i, 1]
    y1 = boxes_ref[i, 2]
    x2 = boxes_ref[i, 3]
    y2 = boxes_ref[i, 4]
    roi_w = jnp.maximum(x2 - x1, 1.0)
    roi_h = jnp.maximum(y2 - y1, 1.0)
    bin_h = roi_h / _OUT
    bin_w = roi_w / _OUT

    ys0 = jnp.clip(y1 + 0.5 * bin_h, 0.0, H - 1.0)
    xs0 = jnp.clip(x1 + 0.5 * bin_w, 0.0, W - 1.0)
    sy = jnp.minimum(
        (jnp.floor(ys0).astype(jnp.int32) // 8) * 8, H - _WIN_Y
    )
    sx = jnp.minimum(
        (jnp.floor(xs0).astype(jnp.int32) // 128) * 128, W - _WIN_X
    )

    jf = jax.lax.broadcasted_iota(jnp.int32, (_OUT, 1), 0).astype(jnp.float32)
    ys = jnp.clip(y1 + (jf + 0.5) * bin_h, 0.0, H - 1.0)
    xs = jnp.clip(x1 + (jf + 0.5) * bin_w, 0.0, W - 1.0)
    y0f = jnp.floor(ys)
    x0f = jnp.floor(xs)
    wy = ys - y0f
    wx = xs - x0f
    y0 = y0f.astype(jnp.int32)
    x0 = x0f.astype(jnp.int32)
    y1i = jnp.minimum(y0 + 1, H - 1)
    x1i = jnp.minimum(x0 + 1, W - 1)

    y0r = y0 - sy
    y1r = y1i - sy
    x0r = x0 - sx
    x1r = x1i - sx

    kky = jax.lax.broadcasted_iota(jnp.int32, (_OUT, _WIN_Y), 1)
    kkx = jax.lax.broadcasted_iota(jnp.int32, (_OUT, _WIN_X), 1)
    w_y = (kky == y0r).astype(jnp.float32) * (1.0 - wy) + (
        kky == y1r
    ).astype(jnp.float32) * wy
    w_x = (kkx == x0r).astype(jnp.float32) * (1.0 - wx) + (
        kkx == x1r
    ).astype(jnp.float32) * wx

    win0 = flows_ref[b, 0, pl.ds(sy, _WIN_Y), pl.ds(sx, _WIN_X)]
    win1 = flows_ref[b, 1, pl.ds(sy, _WIN_Y), pl.ds(sx, _WIN_X)]

    dn_rows = (((1,), (0,)), ((), ()))
    dn_cols = (((1,), (1,)), ((), ()))
    hp = jax.lax.Precision.DEFAULT
    ty0 = jax.lax.dot_general(
        w_y, win0, dn_rows, precision=hp, preferred_element_type=jnp.float32
    )
    ty1 = jax.lax.dot_general(
        w_y, win1, dn_rows, precision=hp, preferred_element_type=jnp.float32
    )
    u = jax.lax.dot_general(
        ty0, w_x, dn_cols, precision=hp, preferred_element_type=jnp.float32
    )
    v = jax.lax.dot_general(
        ty1, w_x, dn_cols, precision=hp, preferred_element_type=jnp.float32
    )

    mag = jnp.sqrt(u * u + v * v)

    su = u >= 0.0
    sv = v >= 0.0
    sm = jnp.abs(u) >= jnp.abs(v)
    nsu = jnp.logical_not(su)
    nsv = jnp.logical_not(sv)
    nsm = jnp.logical_not(sm)
    masks = [
        nsu & nsv & nsm,
        nsu & nsv & sm,
        nsu & sv & sm,
        nsu & sv & nsm,
        su & sv & nsm,
        su & sv & sm,
        su & nsv & sm,
        su & nsv & nsm,
    ]
    zeros = jnp.zeros_like(mag)
    vals = []
    for m in masks:
        s = jnp.sum(jnp.where(m, mag, zeros))
        c = jnp.sum(m.astype(jnp.float32))
        vals.append(jnp.where(c > 0.0, s / jnp.maximum(c, 1.0), 0.0))
    out_ref[i, :] = jnp.stack(vals)


def kernel(flows, boxes):
    K = boxes.shape[0]
    return pl.pallas_call(
        _ve_kernel,
        grid=(K,),
        in_specs=[
            pl.BlockSpec(memory_space=pltpu.SMEM),
            pl.BlockSpec(
                flows.shape, lambda i: (0, 0, 0, 0), memory_space=pltpu.VMEM
            ),
        ],
        out_specs=pl.BlockSpec((K, _NBINS), lambda i: (0, 0)),
        out_shape=jax.ShapeDtypeStruct((K, _NBINS), jnp.float32),
        compiler_params=pltpu.CompilerParams(
            dimension_semantics=("arbitrary",),
        ),
    )(boxes, flows)


# speedup vs baseline: 84.4893x; 1.2261x over previous
import jax
import jax.numpy as jnp
from jax.experimental import pallas as pl
from jax.experimental.pallas import tpu as pltpu

_OUT = 224
_WIN_Y = 240
_WIN_X = 384
_NBINS = 8
_UNROLL = 2


def _one_roi(boxes_ref, flows_ref, out_ref, i):
    H = flows_ref.shape[2]
    W = flows_ref.shape[3]

    b = boxes_ref[i, 0].astype(jnp.int32)
    x1 = boxes_ref[i, 1]
    y1 = boxes_ref[i, 2]
    x2 = boxes_ref[i, 3]
    y2 = boxes_ref[i, 4]
    roi_w = jnp.maximum(x2 - x1, 1.0)
    roi_h = jnp.maximum(y2 - y1, 1.0)
    bin_h = roi_h / _OUT
    bin_w = roi_w / _OUT

    ys0 = jnp.clip(y1 + 0.5 * bin_h, 0.0, H - 1.0)
    xs0 = jnp.clip(x1 + 0.5 * bin_w, 0.0, W - 1.0)
    sy = jnp.minimum((jnp.floor(ys0).astype(jnp.int32) // 8) * 8, H - _WIN_Y)
    sx = jnp.minimum(
        (jnp.floor(xs0).astype(jnp.int32) // 128) * 128, W - _WIN_X
    )

    jf = jax.lax.broadcasted_iota(jnp.int32, (_OUT, 1), 0).astype(jnp.float32)
    ys = jnp.clip(y1 + (jf + 0.5) * bin_h, 0.0, H - 1.0)
    xs = jnp.clip(x1 + (jf + 0.5) * bin_w, 0.0, W - 1.0)
    y0f = jnp.floor(ys)
    x0f = jnp.floor(xs)
    wy = ys - y0f
    wx = xs - x0f
    y0 = y0f.astype(jnp.int32)
    x0 = x0f.astype(jnp.int32)
    y1i = jnp.minimum(y0 + 1, H - 1)
    x1i = jnp.minimum(x0 + 1, W - 1)

    y0r = y0 - sy
    y1r = y1i - sy
    x0r = x0 - sx
    x1r = x1i - sx

    kky = jax.lax.broadcasted_iota(jnp.int32, (_OUT, _WIN_Y), 1)
    kkx = jax.lax.broadcasted_iota(jnp.int32, (_OUT, _WIN_X), 1)
    w_y = (kky == y0r).astype(jnp.float32) * (1.0 - wy) + (
        kky == y1r
    ).astype(jnp.float32) * wy
    w_x = (kkx == x0r).astype(jnp.float32) * (1.0 - wx) + (
        kkx == x1r
    ).astype(jnp.float32) * wx

    win0 = flows_ref[b, 0, pl.ds(sy, _WIN_Y), pl.ds(sx, _WIN_X)]
    win1 = flows_ref[b, 1, pl.ds(sy, _WIN_Y), pl.ds(sx, _WIN_X)]

    dn_rows = (((1,), (0,)), ((), ()))
    dn_cols = (((1,), (1,)), ((), ()))
    hp = jax.lax.Precision.DEFAULT
    ty0 = jax.lax.dot_general(
        w_y, win0, dn_rows, precision=hp, preferred_element_type=jnp.float32
    )
    ty1 = jax.lax.dot_general(
        w_y, win1, dn_rows, precision=hp, preferred_element_type=jnp.float32
    )
    u = jax.lax.dot_general(
        ty0, w_x, dn_cols, precision=hp, preferred_element_type=jnp.float32
    )
    v = jax.lax.dot_general(
        ty1, w_x, dn_cols, precision=hp, preferred_element_type=jnp.float32
    )

    mag = jnp.sqrt(u * u + v * v)

    su = u >= 0.0
    sv = v >= 0.0
    sm = jnp.abs(u) >= jnp.abs(v)
    nsu = jnp.logical_not(su)
    nsv = jnp.logical_not(sv)
    nsm = jnp.logical_not(sm)
    masks = [
        nsu & nsv & nsm,
        nsu & nsv & sm,
        nsu & sv & sm,
        nsu & sv & nsm,
        su & sv & nsm,
        su & sv & sm,
        su & nsv & sm,
        su & nsv & nsm,
    ]
    zeros = jnp.zeros_like(mag)
    vals = []
    for m in masks:
        s = jnp.sum(jnp.where(m, mag, zeros))
        c = jnp.sum(m.astype(jnp.float32))
        vals.append(jnp.where(c > 0.0, s / jnp.maximum(c, 1.0), 0.0))
    out_ref[i, :] = jnp.stack(vals)


def _ve_kernel(boxes_ref, flows_ref, out_ref):
    g = pl.program_id(0)
    for k in range(_UNROLL):
        _one_roi(boxes_ref, flows_ref, out_ref, g * _UNROLL + k)


def kernel(flows, boxes):
    K = boxes.shape[0]
    return pl.pallas_call(
        _ve_kernel,
        grid=(K // _UNROLL,),
        in_specs=[
            pl.BlockSpec(memory_space=pltpu.SMEM),
            pl.BlockSpec(
                flows.shape, lambda i: (0, 0, 0, 0), memory_space=pltpu.VMEM
            ),
        ],
        out_specs=pl.BlockSpec((K, _NBINS), lambda i: (0, 0)),
        out_shape=jax.ShapeDtypeStruct((K, _NBINS), jnp.float32),
        compiler_params=pltpu.CompilerParams(
            dimension_semantics=("arbitrary",),
        ),
    )(boxes, flows)


# speedup vs baseline: 93.7013x; 1.1090x over previous
import jax
import jax.numpy as jnp
from jax.experimental import pallas as pl
from jax.experimental.pallas import tpu as pltpu

_OUT = 224
_WIN_Y = 240
_WIN_X = 384
_NBINS = 8
_UNROLL = 4


def _one_roi(boxes_ref, flows_ref, out_ref, i):
    H = flows_ref.shape[2]
    W = flows_ref.shape[3]

    b = boxes_ref[i, 0].astype(jnp.int32)
    x1 = boxes_ref[i, 1]
    y1 = boxes_ref[i, 2]
    x2 = boxes_ref[i, 3]
    y2 = boxes_ref[i, 4]
    roi_w = jnp.maximum(x2 - x1, 1.0)
    roi_h = jnp.maximum(y2 - y1, 1.0)
    bin_h = roi_h / _OUT
    bin_w = roi_w / _OUT

    ys0 = jnp.clip(y1 + 0.5 * bin_h, 0.0, H - 1.0)
    xs0 = jnp.clip(x1 + 0.5 * bin_w, 0.0, W - 1.0)
    sy = jnp.minimum((jnp.floor(ys0).astype(jnp.int32) // 8) * 8, H - _WIN_Y)
    sx = jnp.minimum(
        (jnp.floor(xs0).astype(jnp.int32) // 128) * 128, W - _WIN_X
    )

    jf = jax.lax.broadcasted_iota(jnp.int32, (_OUT, 1), 0).astype(jnp.float32)
    ys = jnp.clip(y1 + (jf + 0.5) * bin_h, 0.0, H - 1.0)
    xs = jnp.clip(x1 + (jf + 0.5) * bin_w, 0.0, W - 1.0)
    y0f = jnp.floor(ys)
    x0f = jnp.floor(xs)
    wy = ys - y0f
    wx = xs - x0f
    y0 = y0f.astype(jnp.int32)
    x0 = x0f.astype(jnp.int32)
    y1i = jnp.minimum(y0 + 1, H - 1)
    x1i = jnp.minimum(x0 + 1, W - 1)

    y0r = y0 - sy
    y1r = y1i - sy
    x0r = x0 - sx
    x1r = x1i - sx

    kky = jax.lax.broadcasted_iota(jnp.int32, (_OUT, _WIN_Y), 1)
    kkx = jax.lax.broadcasted_iota(jnp.int32, (_OUT, _WIN_X), 1)
    w_y = (kky == y0r).astype(jnp.float32) * (1.0 - wy) + (
        kky == y1r
    ).astype(jnp.float32) * wy
    w_x = (kkx == x0r).astype(jnp.float32) * (1.0 - wx) + (
        kkx == x1r
    ).astype(jnp.float32) * wx

    win0 = flows_ref[b, 0, pl.ds(sy, _WIN_Y), pl.ds(sx, _WIN_X)]
    win1 = flows_ref[b, 1, pl.ds(sy, _WIN_Y), pl.ds(sx, _WIN_X)]

    dn_rows = (((1,), (0,)), ((), ()))
    dn_cols = (((1,), (1,)), ((), ()))
    hp = jax.lax.Precision.DEFAULT
    ty0 = jax.lax.dot_general(
        w_y, win0, dn_rows, precision=hp, preferred_element_type=jnp.float32
    )
    ty1 = jax.lax.dot_general(
        w_y, win1, dn_rows, precision=hp, preferred_element_type=jnp.float32
    )
    u = jax.lax.dot_general(
        ty0, w_x, dn_cols, precision=hp, preferred_element_type=jnp.float32
    )
    v = jax.lax.dot_general(
        ty1, w_x, dn_cols, precision=hp, preferred_element_type=jnp.float32
    )

    mag = jnp.sqrt(u * u + v * v)

    su = u >= 0.0
    sv = v >= 0.0
    sm = jnp.abs(u) >= jnp.abs(v)
    nsu = jnp.logical_not(su)
    nsv = jnp.logical_not(sv)
    nsm = jnp.logical_not(sm)
    masks = [
        nsu & nsv & nsm,
        nsu & nsv & sm,
        nsu & sv & sm,
        nsu & sv & nsm,
        su & sv & nsm,
        su & sv & sm,
        su & nsv & sm,
        su & nsv & nsm,
    ]
    zeros = jnp.zeros_like(mag)
    vals = []
    for m in masks:
        s = jnp.sum(jnp.where(m, mag, zeros))
        c = jnp.sum(m.astype(jnp.float32))
        vals.append(jnp.where(c > 0.0, s / jnp.maximum(c, 1.0), 0.0))
    out_ref[i, :] = jnp.stack(vals)


def _ve_kernel(boxes_ref, flows_ref, out_ref):
    g = pl.program_id(0)
    for k in range(_UNROLL):
        _one_roi(boxes_ref, flows_ref, out_ref, g * _UNROLL + k)


def kernel(flows, boxes):
    K = boxes.shape[0]
    return pl.pallas_call(
        _ve_kernel,
        grid=(K // _UNROLL,),
        in_specs=[
            pl.BlockSpec(memory_space=pltpu.SMEM),
            pl.BlockSpec(
                flows.shape, lambda i: (0, 0, 0, 0), memory_space=pltpu.VMEM
            ),
        ],
        out_specs=pl.BlockSpec((K, _NBINS), lambda i: (0, 0)),
        out_shape=jax.ShapeDtypeStruct((K, _NBINS), jnp.float32),
        compiler_params=pltpu.CompilerParams(
            dimension_semantics=("arbitrary",),
        ),
    )(boxes, flows)


# speedup vs baseline: 97.8799x; 1.0446x over previous
import jax
import jax.numpy as jnp
from jax.experimental import pallas as pl
from jax.experimental.pallas import tpu as pltpu

_OUT = 224
_WIN_Y = 240
_WIN_X = 384
_NBINS = 8
_UNROLL = 8


def _one_roi(boxes_ref, flows_ref, out_ref, i):
    H = flows_ref.shape[2]
    W = flows_ref.shape[3]

    b = boxes_ref[i, 0].astype(jnp.int32)
    x1 = boxes_ref[i, 1]
    y1 = boxes_ref[i, 2]
    x2 = boxes_ref[i, 3]
    y2 = boxes_ref[i, 4]
    roi_w = jnp.maximum(x2 - x1, 1.0)
    roi_h = jnp.maximum(y2 - y1, 1.0)
    bin_h = roi_h / _OUT
    bin_w = roi_w / _OUT

    ys0 = jnp.clip(y1 + 0.5 * bin_h, 0.0, H - 1.0)
    xs0 = jnp.clip(x1 + 0.5 * bin_w, 0.0, W - 1.0)
    sy = jnp.minimum((jnp.floor(ys0).astype(jnp.int32) // 8) * 8, H - _WIN_Y)
    sx = jnp.minimum(
        (jnp.floor(xs0).astype(jnp.int32) // 128) * 128, W - _WIN_X
    )

    jf = jax.lax.broadcasted_iota(jnp.int32, (_OUT, 1), 0).astype(jnp.float32)
    ys = jnp.clip(y1 + (jf + 0.5) * bin_h, 0.0, H - 1.0)
    xs = jnp.clip(x1 + (jf + 0.5) * bin_w, 0.0, W - 1.0)
    y0f = jnp.floor(ys)
    x0f = jnp.floor(xs)
    wy = ys - y0f
    wx = xs - x0f
    y0 = y0f.astype(jnp.int32)
    x0 = x0f.astype(jnp.int32)
    y1i = jnp.minimum(y0 + 1, H - 1)
    x1i = jnp.minimum(x0 + 1, W - 1)

    y0r = y0 - sy
    y1r = y1i - sy
    x0r = x0 - sx
    x1r = x1i - sx

    kky = jax.lax.broadcasted_iota(jnp.int32, (_OUT, _WIN_Y), 1)
    kkx = jax.lax.broadcasted_iota(jnp.int32, (_OUT, _WIN_X), 1)
    w_y = (kky == y0r).astype(jnp.float32) * (1.0 - wy) + (
        kky == y1r
    ).astype(jnp.float32) * wy
    w_x = (kkx == x0r).astype(jnp.float32) * (1.0 - wx) + (
        kkx == x1r
    ).astype(jnp.float32) * wx

    win0 = flows_ref[b, 0, pl.ds(sy, _WIN_Y), pl.ds(sx, _WIN_X)]
    win1 = flows_ref[b, 1, pl.ds(sy, _WIN_Y), pl.ds(sx, _WIN_X)]

    dn_rows = (((1,), (0,)), ((), ()))
    dn_cols = (((1,), (1,)), ((), ()))
    hp = jax.lax.Precision.DEFAULT
    ty0 = jax.lax.dot_general(
        w_y, win0, dn_rows, precision=hp, preferred_element_type=jnp.float32
    )
    ty1 = jax.lax.dot_general(
        w_y, win1, dn_rows, precision=hp, preferred_element_type=jnp.float32
    )
    u = jax.lax.dot_general(
        ty0, w_x, dn_cols, precision=hp, preferred_element_type=jnp.float32
    )
    v = jax.lax.dot_general(
        ty1, w_x, dn_cols, precision=hp, preferred_element_type=jnp.float32
    )

    mag = jnp.sqrt(u * u + v * v)

    su = u >= 0.0
    sv = v >= 0.0
    sm = jnp.abs(u) >= jnp.abs(v)
    nsu = jnp.logical_not(su)
    nsv = jnp.logical_not(sv)
    nsm = jnp.logical_not(sm)
    masks = [
        nsu & nsv & nsm,
        nsu & nsv & sm,
        nsu & sv & sm,
        nsu & sv & nsm,
        su & sv & nsm,
        su & sv & sm,
        su & nsv & sm,
        su & nsv & nsm,
    ]
    zeros = jnp.zeros_like(mag)
    vals = []
    for m in masks:
        s = jnp.sum(jnp.where(m, mag, zeros))
        c = jnp.sum(m.astype(jnp.float32))
        vals.append(jnp.where(c > 0.0, s / jnp.maximum(c, 1.0), 0.0))
    out_ref[i, :] = jnp.stack(vals)


def _ve_kernel(boxes_ref, flows_ref, out_ref):
    g = pl.program_id(0)
    for k in range(_UNROLL):
        _one_roi(boxes_ref, flows_ref, out_ref, g * _UNROLL + k)


def kernel(flows, boxes):
    K = boxes.shape[0]
    return pl.pallas_call(
        _ve_kernel,
        grid=(K // _UNROLL,),
        in_specs=[
            pl.BlockSpec(memory_space=pltpu.SMEM),
            pl.BlockSpec(
                flows.shape, lambda i: (0, 0, 0, 0), memory_space=pltpu.VMEM
            ),
        ],
        out_specs=pl.BlockSpec((K, _NBINS), lambda i: (0, 0)),
        out_shape=jax.ShapeDtypeStruct((K, _NBINS), jnp.float32),
        compiler_params=pltpu.CompilerParams(
            dimension_semantics=("arbitrary",),
        ),
    )(boxes, flows)


# speedup vs baseline: 149.0427x; 1.5227x over previous
import itertools

import jax
import jax.numpy as jnp
from jax.experimental import pallas as pl
from jax.experimental.pallas import tpu as pltpu

_OUT = 224
_WIN_Y = 240
_WIN_X = 384
_NBINS = 8
_UNROLL = 8


def _one_roi(boxes_ref, flows_ref, out_ref, i):
    H = flows_ref.shape[2]
    W = flows_ref.shape[3]

    b = boxes_ref[i, 0].astype(jnp.int32)
    x1 = boxes_ref[i, 1]
    y1 = boxes_ref[i, 2]
    x2 = boxes_ref[i, 3]
    y2 = boxes_ref[i, 4]
    roi_w = jnp.maximum(x2 - x1, 1.0)
    roi_h = jnp.maximum(y2 - y1, 1.0)
    bin_h = roi_h / _OUT
    bin_w = roi_w / _OUT

    ys0 = jnp.clip(y1 + 0.5 * bin_h, 0.0, H - 1.0)
    xs0 = jnp.clip(x1 + 0.5 * bin_w, 0.0, W - 1.0)
    sy = jnp.minimum((jnp.floor(ys0).astype(jnp.int32) // 8) * 8, H - _WIN_Y)
    sx = jnp.minimum(
        (jnp.floor(xs0).astype(jnp.int32) // 128) * 128, W - _WIN_X
    )

    jf = jax.lax.broadcasted_iota(jnp.int32, (_OUT, 1), 0).astype(jnp.float32)
    ys = jnp.clip(y1 + (jf + 0.5) * bin_h, 0.0, H - 1.0)
    xs = jnp.clip(x1 + (jf + 0.5) * bin_w, 0.0, W - 1.0)
    kky = jax.lax.broadcasted_iota(jnp.int32, (_OUT, _WIN_Y), 1).astype(
        jnp.float32
    )
    kkx = jax.lax.broadcasted_iota(jnp.int32, (_OUT, _WIN_X), 1).astype(
        jnp.float32
    )
    ycr = ys - sy.astype(jnp.float32)
    xcr = xs - sx.astype(jnp.float32)
    w_y = jnp.maximum(1.0 - jnp.abs(kky - ycr), 0.0)
    w_x = jnp.maximum(1.0 - jnp.abs(kkx - xcr), 0.0)

    win0 = flows_ref[b, 0, pl.ds(sy, _WIN_Y), pl.ds(sx, _WIN_X)]
    win1 = flows_ref[b, 1, pl.ds(sy, _WIN_Y), pl.ds(sx, _WIN_X)]

    dn_rows = (((1,), (0,)), ((), ()))
    dn_cols = (((1,), (1,)), ((), ()))
    hp = jax.lax.Precision.DEFAULT
    ty0 = jax.lax.dot_general(
        w_y, win0, dn_rows, precision=hp, preferred_element_type=jnp.float32
    )
    ty1 = jax.lax.dot_general(
        w_y, win1, dn_rows, precision=hp, preferred_element_type=jnp.float32
    )
    u = jax.lax.dot_general(
        ty0, w_x, dn_cols, precision=hp, preferred_element_type=jnp.float32
    )
    v = jax.lax.dot_general(
        ty1, w_x, dn_cols, precision=hp, preferred_element_type=jnp.float32
    )

    mag = jnp.sqrt(u * u + v * v)

    b2 = jnp.where(u >= 0.0, 1.0, 0.0)
    bv = jnp.where(v >= 0.0, 1.0, 0.0)
    bm = jnp.where(jnp.abs(u) >= jnp.abs(v), 1.0, 0.0)
    b1 = jnp.abs(b2 - bv)
    b0 = jnp.abs(b1 - bm)
    bits = (b0, b1, b2)

    m0 = mag * b0
    m1 = mag * b1
    m2 = mag * b2
    m01 = m0 * b1
    m02 = m0 * b2
    m12 = m1 * b2
    m012 = m01 * b2
    p01 = b0 * b1
    p02 = b0 * b2
    p12 = b1 * b2
    p012 = p01 * b2
    um = {
        (): jnp.sum(mag),
        (0,): jnp.sum(m0),
        (1,): jnp.sum(m1),
        (2,): jnp.sum(m2),
        (0, 1): jnp.sum(m01),
        (0, 2): jnp.sum(m02),
        (1, 2): jnp.sum(m12),
        (0, 1, 2): jnp.sum(m012),
    }
    uc = {
        (): jnp.float32(_OUT * _OUT),
        (0,): jnp.sum(b0),
        (1,): jnp.sum(b1),
        (2,): jnp.sum(b2),
        (0, 1): jnp.sum(p01),
        (0, 2): jnp.sum(p02),
        (1, 2): jnp.sum(p12),
        (0, 1, 2): jnp.sum(p012),
    }
    vals = []
    for bidx in range(_NBINS):
        beta = (bidx & 1, (bidx >> 1) & 1, (bidx >> 2) & 1)
        ones = tuple(t for t in range(3) if beta[t])
        zeros_ = tuple(t for t in range(3) if not beta[t])
        s = None
        c = None
        for r in range(len(zeros_) + 1):
            for extra in itertools.combinations(zeros_, r):
                key = tuple(sorted(ones + extra))
                sgn = -1.0 if (len(extra) % 2) else 1.0
                term_s = sgn * um[key]
                term_c = sgn * uc[key]
                s = term_s if s is None else s + term_s
                c = term_c if c is None else c + term_c
        vals.append(jnp.where(c > 0.0, s / jnp.maximum(c, 1.0), 0.0))
    out_ref[i, :] = jnp.stack(vals)


def _ve_kernel(boxes_ref, flows_ref, out_ref):
    g = pl.program_id(0)
    for k in range(_UNROLL):
        _one_roi(boxes_ref, flows_ref, out_ref, g * _UNROLL + k)


def kernel(flows, boxes):
    K = boxes.shape[0]
    return pl.pallas_call(
        _ve_kernel,
        grid=(K // _UNROLL,),
        in_specs=[
            pl.BlockSpec(memory_space=pltpu.SMEM),
            pl.BlockSpec(
                flows.shape, lambda i: (0, 0, 0, 0), memory_space=pltpu.VMEM
            ),
        ],
        out_specs=pl.BlockSpec((K, _NBINS), lambda i: (0, 0)),
        out_shape=jax.ShapeDtypeStruct((K, _NBINS), jnp.float32),
        compiler_params=pltpu.CompilerParams(
            dimension_semantics=("arbitrary",),
        ),
    )(boxes, flows)


# speedup vs baseline: 149.3214x; 1.0019x over previous
import itertools

import jax
import jax.numpy as jnp
from jax.experimental import pallas as pl
from jax.experimental.pallas import tpu as pltpu

_OUT = 224
_WIN_Y = 240
_WIN_X = 384
_NBINS = 8
_UNROLL = 8


def _one_roi(boxes_ref, flows_ref, out_ref, i):
    H = flows_ref.shape[2]
    W = flows_ref.shape[3]

    b = boxes_ref[i, 0].astype(jnp.int32)
    x1 = boxes_ref[i, 1]
    y1 = boxes_ref[i, 2]
    x2 = boxes_ref[i, 3]
    y2 = boxes_ref[i, 4]
    roi_w = jnp.maximum(x2 - x1, 1.0)
    roi_h = jnp.maximum(y2 - y1, 1.0)
    bin_h = roi_h / _OUT
    bin_w = roi_w / _OUT

    ys0 = jnp.clip(y1 + 0.5 * bin_h, 0.0, H - 1.0)
    xs0 = jnp.clip(x1 + 0.5 * bin_w, 0.0, W - 1.0)
    sy = jnp.minimum((jnp.floor(ys0).astype(jnp.int32) // 8) * 8, H - _WIN_Y)
    sx = jnp.minimum(
        (jnp.floor(xs0).astype(jnp.int32) // 128) * 128, W - _WIN_X
    )

    jf = jax.lax.broadcasted_iota(jnp.int32, (_OUT, 1), 0).astype(jnp.float32)
    ys = jnp.clip(y1 + (jf + 0.5) * bin_h, 0.0, H - 1.0)
    xs = jnp.clip(x1 + (jf + 0.5) * bin_w, 0.0, W - 1.0)
    kky = jax.lax.broadcasted_iota(jnp.int32, (_OUT, _WIN_Y), 1).astype(
        jnp.float32
    )
    kkx = jax.lax.broadcasted_iota(jnp.int32, (_OUT, _WIN_X), 1).astype(
        jnp.float32
    )
    ycr = ys - sy.astype(jnp.float32)
    xcr = xs - sx.astype(jnp.float32)
    w_y = jnp.maximum(1.0 - jnp.abs(kky - ycr), 0.0)
    w_x = jnp.maximum(1.0 - jnp.abs(kkx - xcr), 0.0)

    win0 = flows_ref[b, 0, pl.ds(sy, _WIN_Y), pl.ds(sx, _WIN_X)]
    win1 = flows_ref[b, 1, pl.ds(sy, _WIN_Y), pl.ds(sx, _WIN_X)]

    dn_rows = (((1,), (0,)), ((), ()))
    dn_cols = (((1,), (1,)), ((), ()))
    hp = jax.lax.Precision.DEFAULT
    ty0 = jax.lax.dot_general(
        w_y, win0, dn_rows, precision=hp, preferred_element_type=jnp.float32
    )
    ty1 = jax.lax.dot_general(
        w_y, win1, dn_rows, precision=hp, preferred_element_type=jnp.float32
    )
    u = jax.lax.dot_general(
        ty0, w_x, dn_cols, precision=hp, preferred_element_type=jnp.float32
    )
    v = jax.lax.dot_general(
        ty1, w_x, dn_cols, precision=hp, preferred_element_type=jnp.float32
    )

    mag = jnp.sqrt(u * u + v * v)

    b2 = jnp.where(u >= 0.0, 1.0, 0.0)
    bv = jnp.where(v >= 0.0, 1.0, 0.0)
    bm = jnp.where(jnp.abs(u) >= jnp.abs(v), 1.0, 0.0)
    b1 = jnp.abs(b2 - bv)
    b0 = jnp.abs(b1 - bm)
    bits = (b0, b1, b2)

    m0 = mag * b0
    m1 = mag * b1
    m2 = mag * b2
    m01 = m0 * b1
    m02 = m0 * b2
    m12 = m1 * b2
    m012 = m01 * b2
    p01 = b0 * b1
    p02 = b0 * b2
    p12 = b1 * b2
    p012 = p01 * b2
    um = {
        (): jnp.sum(mag),
        (0,): jnp.sum(m0),
        (1,): jnp.sum(m1),
        (2,): jnp.sum(m2),
        (0, 1): jnp.sum(m01),
        (0, 2): jnp.sum(m02),
        (1, 2): jnp.sum(m12),
        (0, 1, 2): jnp.sum(m012),
    }
    uc = {
        (): jnp.float32(_OUT * _OUT),
        (0,): jnp.sum(b0),
        (1,): jnp.sum(b1),
        (2,): jnp.sum(b2),
        (0, 1): jnp.sum(p01),
        (0, 2): jnp.sum(p02),
        (1, 2): jnp.sum(p12),
        (0, 1, 2): jnp.sum(p012),
    }
    vals = []
    for bidx in range(_NBINS):
        beta = (bidx & 1, (bidx >> 1) & 1, (bidx >> 2) & 1)
        ones = tuple(t for t in range(3) if beta[t])
        zeros_ = tuple(t for t in range(3) if not beta[t])
        s = None
        c = None
        for r in range(len(zeros_) + 1):
            for extra in itertools.combinations(zeros_, r):
                key = tuple(sorted(ones + extra))
                sgn = -1.0 if (len(extra) % 2) else 1.0
                term_s = sgn * um[key]
                term_c = sgn * uc[key]
                s = term_s if s is None else s + term_s
                c = term_c if c is None else c + term_c
        vals.append(jnp.where(c > 0.0, s / jnp.maximum(c, 1.0), 0.0))
    out_ref[i, :] = jnp.stack(vals)


def _ve_kernel(boxes_ref, flows_ref, out_ref):
    g = pl.program_id(0)
    for k in range(_UNROLL):
        _one_roi(boxes_ref, flows_ref, out_ref, g * _UNROLL + k)


def kernel(flows, boxes):
    K = boxes.shape[0]
    return pl.pallas_call(
        _ve_kernel,
        grid=(K // _UNROLL,),
        in_specs=[
            pl.BlockSpec(memory_space=pltpu.SMEM),
            pl.BlockSpec(
                flows.shape, lambda i: (0, 0, 0, 0), memory_space=pltpu.VMEM
            ),
        ],
        out_specs=pl.BlockSpec((K, _NBINS), lambda i: (0, 0)),
        out_shape=jax.ShapeDtypeStruct((K, _NBINS), jnp.float32),
        compiler_params=pltpu.CompilerParams(
            dimension_semantics=("parallel",),
        ),
    )(boxes, flows)


# speedup vs baseline: 154.4836x; 1.0346x over previous
import itertools

import jax
import jax.numpy as jnp
from jax.experimental import pallas as pl
from jax.experimental.pallas import tpu as pltpu

_OUT = 224
_WIN_Y = 240
_WIN_X = 384
_NBINS = 8
_UNROLL = 16


def _one_roi(boxes_ref, flows_ref, out_ref, i):
    H = flows_ref.shape[2]
    W = flows_ref.shape[3]

    b = boxes_ref[i, 0].astype(jnp.int32)
    x1 = boxes_ref[i, 1]
    y1 = boxes_ref[i, 2]
    x2 = boxes_ref[i, 3]
    y2 = boxes_ref[i, 4]
    roi_w = jnp.maximum(x2 - x1, 1.0)
    roi_h = jnp.maximum(y2 - y1, 1.0)
    bin_h = roi_h / _OUT
    bin_w = roi_w / _OUT

    ys0 = jnp.clip(y1 + 0.5 * bin_h, 0.0, H - 1.0)
    xs0 = jnp.clip(x1 + 0.5 * bin_w, 0.0, W - 1.0)
    sy = jnp.minimum((jnp.floor(ys0).astype(jnp.int32) // 8) * 8, H - _WIN_Y)
    sx = jnp.minimum(
        (jnp.floor(xs0).astype(jnp.int32) // 128) * 128, W - _WIN_X
    )

    jf = jax.lax.broadcasted_iota(jnp.int32, (_OUT, 1), 0).astype(jnp.float32)
    ys = jnp.clip(y1 + (jf + 0.5) * bin_h, 0.0, H - 1.0)
    xs = jnp.clip(x1 + (jf + 0.5) * bin_w, 0.0, W - 1.0)
    kky = jax.lax.broadcasted_iota(jnp.int32, (_OUT, _WIN_Y), 1).astype(
        jnp.float32
    )
    kkx = jax.lax.broadcasted_iota(jnp.int32, (_OUT, _WIN_X), 1).astype(
        jnp.float32
    )
    ycr = ys - sy.astype(jnp.float32)
    xcr = xs - sx.astype(jnp.float32)
    w_y = jnp.maximum(1.0 - jnp.abs(kky - ycr), 0.0)
    w_x = jnp.maximum(1.0 - jnp.abs(kkx - xcr), 0.0)

    win0 = flows_ref[b, 0, pl.ds(sy, _WIN_Y), pl.ds(sx, _WIN_X)]
    win1 = flows_ref[b, 1, pl.ds(sy, _WIN_Y), pl.ds(sx, _WIN_X)]

    dn_rows = (((1,), (0,)), ((), ()))
    dn_cols = (((1,), (1,)), ((), ()))
    hp = jax.lax.Precision.DEFAULT
    ty0 = jax.lax.dot_general(
        w_y, win0, dn_rows, precision=hp, preferred_element_type=jnp.float32
    )
    ty1 = jax.lax.dot_general(
        w_y, win1, dn_rows, precision=hp, preferred_element_type=jnp.float32
    )
    u = jax.lax.dot_general(
        ty0, w_x, dn_cols, precision=hp, preferred_element_type=jnp.float32
    )
    v = jax.lax.dot_general(
        ty1, w_x, dn_cols, precision=hp, preferred_element_type=jnp.float32
    )

    mag = jnp.sqrt(u * u + v * v)

    b2 = jnp.where(u >= 0.0, 1.0, 0.0)
    bv = jnp.where(v >= 0.0, 1.0, 0.0)
    bm = jnp.where(jnp.abs(u) >= jnp.abs(v), 1.0, 0.0)
    b1 = jnp.abs(b2 - bv)
    b0 = jnp.abs(b1 - bm)
    bits = (b0, b1, b2)

    m0 = mag * b0
    m1 = mag * b1
    m2 = mag * b2
    m01 = m0 * b1
    m02 = m0 * b2
    m12 = m1 * b2
    m012 = m01 * b2
    p01 = b0 * b1
    p02 = b0 * b2
    p12 = b1 * b2
    p012 = p01 * b2
    um = {
        (): jnp.sum(mag),
        (0,): jnp.sum(m0),
        (1,): jnp.sum(m1),
        (2,): jnp.sum(m2),
        (0, 1): jnp.sum(m01),
        (0, 2): jnp.sum(m02),
        (1, 2): jnp.sum(m12),
        (0, 1, 2): jnp.sum(m012),
    }
    uc = {
        (): jnp.float32(_OUT * _OUT),
        (0,): jnp.sum(b0),
        (1,): jnp.sum(b1),
        (2,): jnp.sum(b2),
        (0, 1): jnp.sum(p01),
        (0, 2): jnp.sum(p02),
        (1, 2): jnp.sum(p12),
        (0, 1, 2): jnp.sum(p012),
    }
    vals = []
    for bidx in range(_NBINS):
        beta = (bidx & 1, (bidx >> 1) & 1, (bidx >> 2) & 1)
        ones = tuple(t for t in range(3) if beta[t])
        zeros_ = tuple(t for t in range(3) if not beta[t])
        s = None
        c = None
        for r in range(len(zeros_) + 1):
            for extra in itertools.combinations(zeros_, r):
                key = tuple(sorted(ones + extra))
                sgn = -1.0 if (len(extra) % 2) else 1.0
                term_s = sgn * um[key]
                term_c = sgn * uc[key]
                s = term_s if s is None else s + term_s
                c = term_c if c is None else c + term_c
        vals.append(jnp.where(c > 0.0, s / jnp.maximum(c, 1.0), 0.0))
    out_ref[i, :] = jnp.stack(vals)


def _ve_kernel(boxes_ref, flows_ref, out_ref):
    g = pl.program_id(0)
    for k in range(_UNROLL):
        _one_roi(boxes_ref, flows_ref, out_ref, g * _UNROLL + k)


def kernel(flows, boxes):
    K = boxes.shape[0]
    return pl.pallas_call(
        _ve_kernel,
        grid=(K // _UNROLL,),
        in_specs=[
            pl.BlockSpec(memory_space=pltpu.SMEM),
            pl.BlockSpec(
                flows.shape, lambda i: (0, 0, 0, 0), memory_space=pltpu.VMEM
            ),
        ],
        out_specs=pl.BlockSpec((K, _NBINS), lambda i: (0, 0)),
        out_shape=jax.ShapeDtypeStruct((K, _NBINS), jnp.float32),
        compiler_params=pltpu.CompilerParams(
            dimension_semantics=("arbitrary",),
        ),
    )(boxes, flows)


# speedup vs baseline: 157.8080x; 1.0215x over previous
import itertools

import jax
import jax.numpy as jnp
from jax.experimental import pallas as pl
from jax.experimental.pallas import tpu as pltpu

_OUT = 224
_WIN_Y = 240
_WIN_X = 384
_NBINS = 8
_UNROLL = 32


def _one_roi(boxes_ref, flows_ref, out_ref, i):
    H = flows_ref.shape[2]
    W = flows_ref.shape[3]

    b = boxes_ref[i, 0].astype(jnp.int32)
    x1 = boxes_ref[i, 1]
    y1 = boxes_ref[i, 2]
    x2 = boxes_ref[i, 3]
    y2 = boxes_ref[i, 4]
    roi_w = jnp.maximum(x2 - x1, 1.0)
    roi_h = jnp.maximum(y2 - y1, 1.0)
    bin_h = roi_h / _OUT
    bin_w = roi_w / _OUT

    ys0 = jnp.clip(y1 + 0.5 * bin_h, 0.0, H - 1.0)
    xs0 = jnp.clip(x1 + 0.5 * bin_w, 0.0, W - 1.0)
    sy = jnp.minimum((jnp.floor(ys0).astype(jnp.int32) // 8) * 8, H - _WIN_Y)
    sx = jnp.minimum(
        (jnp.floor(xs0).astype(jnp.int32) // 128) * 128, W - _WIN_X
    )

    jf = jax.lax.broadcasted_iota(jnp.int32, (_OUT, 1), 0).astype(jnp.float32)
    ys = jnp.clip(y1 + (jf + 0.5) * bin_h, 0.0, H - 1.0)
    xs = jnp.clip(x1 + (jf + 0.5) * bin_w, 0.0, W - 1.0)
    kky = jax.lax.broadcasted_iota(jnp.int32, (_OUT, _WIN_Y), 1).astype(
        jnp.float32
    )
    kkx = jax.lax.broadcasted_iota(jnp.int32, (_OUT, _WIN_X), 1).astype(
        jnp.float32
    )
    ycr = ys - sy.astype(jnp.float32)
    xcr = xs - sx.astype(jnp.float32)
    w_y = jnp.maximum(1.0 - jnp.abs(kky - ycr), 0.0)
    w_x = jnp.maximum(1.0 - jnp.abs(kkx - xcr), 0.0)

    win0 = flows_ref[b, 0, pl.ds(sy, _WIN_Y), pl.ds(sx, _WIN_X)]
    win1 = flows_ref[b, 1, pl.ds(sy, _WIN_Y), pl.ds(sx, _WIN_X)]

    dn_rows = (((1,), (0,)), ((), ()))
    dn_cols = (((1,), (1,)), ((), ()))
    hp = jax.lax.Precision.DEFAULT
    ty0 = jax.lax.dot_general(
        w_y, win0, dn_rows, precision=hp, preferred_element_type=jnp.float32
    )
    ty1 = jax.lax.dot_general(
        w_y, win1, dn_rows, precision=hp, preferred_element_type=jnp.float32
    )
    u = jax.lax.dot_general(
        ty0, w_x, dn_cols, precision=hp, preferred_element_type=jnp.float32
    )
    v = jax.lax.dot_general(
        ty1, w_x, dn_cols, precision=hp, preferred_element_type=jnp.float32
    )

    mag = jnp.sqrt(u * u + v * v)

    b2 = jnp.where(u >= 0.0, 1.0, 0.0)
    bv = jnp.where(v >= 0.0, 1.0, 0.0)
    bm = jnp.where(jnp.abs(u) >= jnp.abs(v), 1.0, 0.0)
    b1 = jnp.abs(b2 - bv)
    b0 = jnp.abs(b1 - bm)
    bits = (b0, b1, b2)

    m0 = mag * b0
    m1 = mag * b1
    m2 = mag * b2
    m01 = m0 * b1
    m02 = m0 * b2
    m12 = m1 * b2
    m012 = m01 * b2
    p01 = b0 * b1
    p02 = b0 * b2
    p12 = b1 * b2
    p012 = p01 * b2
    um = {
        (): jnp.sum(mag),
        (0,): jnp.sum(m0),
        (1,): jnp.sum(m1),
        (2,): jnp.sum(m2),
        (0, 1): jnp.sum(m01),
        (0, 2): jnp.sum(m02),
        (1, 2): jnp.sum(m12),
        (0, 1, 2): jnp.sum(m012),
    }
    uc = {
        (): jnp.float32(_OUT * _OUT),
        (0,): jnp.sum(b0),
        (1,): jnp.sum(b1),
        (2,): jnp.sum(b2),
        (0, 1): jnp.sum(p01),
        (0, 2): jnp.sum(p02),
        (1, 2): jnp.sum(p12),
        (0, 1, 2): jnp.sum(p012),
    }
    vals = []
    for bidx in range(_NBINS):
        beta = (bidx & 1, (bidx >> 1) & 1, (bidx >> 2) & 1)
        ones = tuple(t for t in range(3) if beta[t])
        zeros_ = tuple(t for t in range(3) if not beta[t])
        s = None
        c = None
        for r in range(len(zeros_) + 1):
            for extra in itertools.combinations(zeros_, r):
                key = tuple(sorted(ones + extra))
                sgn = -1.0 if (len(extra) % 2) else 1.0
                term_s = sgn * um[key]
                term_c = sgn * uc[key]
                s = term_s if s is None else s + term_s
                c = term_c if c is None else c + term_c
        vals.append(jnp.where(c > 0.0, s / jnp.maximum(c, 1.0), 0.0))
    out_ref[i, :] = jnp.stack(vals)


def _ve_kernel(boxes_ref, flows_ref, out_ref):
    g = pl.program_id(0)
    for k in range(_UNROLL):
        _one_roi(boxes_ref, flows_ref, out_ref, g * _UNROLL + k)


def kernel(flows, boxes):
    K = boxes.shape[0]
    return pl.pallas_call(
        _ve_kernel,
        grid=(K // _UNROLL,),
        in_specs=[
            pl.BlockSpec(memory_space=pltpu.SMEM),
            pl.BlockSpec(
                flows.shape, lambda i: (0, 0, 0, 0), memory_space=pltpu.VMEM
            ),
        ],
        out_specs=pl.BlockSpec((K, _NBINS), lambda i: (0, 0)),
        out_shape=jax.ShapeDtypeStruct((K, _NBINS), jnp.float32),
        compiler_params=pltpu.CompilerParams(
            dimension_semantics=("arbitrary",),
        ),
    )(boxes, flows)
